# Initial kernel scaffold; baseline (speedup 1.0000x reference)
import jax
import jax.numpy as jnp
from jax.experimental import pallas as pl


def kernel(x):
    raise NotImplementedError("write your pallas kernel here")


# speedup vs baseline: 3.1756x; 3.1756x over previous
import jax
import jax.numpy as jnp
from jax.experimental import pallas as pl

ROWS = 2048


def _select_kernel(x_ref, q_ref, o_ref):
    o_ref[...] = jnp.dot(
        x_ref[...], q_ref[...], preferred_element_type=jnp.float32
    )


def kernel(x):
    b, r, c = x.shape
    n = b * r
    flat = x.reshape(n, c)
    q = (jnp.arange(512)[:, None] == 4 * jnp.arange(128)[None, :]).astype(
        x.dtype
    )
    out = pl.pallas_call(
        _select_kernel,
        grid=(n // ROWS, c // 512),
        in_specs=[
            pl.BlockSpec((ROWS, 512), lambda i, j: (i, j)),
            pl.BlockSpec((512, 128), lambda i, j: (0, 0)),
        ],
        out_specs=pl.BlockSpec((ROWS, 128), lambda i, j: (i, j)),
        out_shape=jax.ShapeDtypeStruct((n, c // 4), x.dtype),
    )(flat, q)
    return out.reshape(b, r, c // 4)


# speedup vs baseline: 3.6015x; 1.1341x over previous
import jax
import jax.numpy as jnp
from jax.experimental import pallas as pl

ROWS = 1024


def _select_kernel(x_ref, q_ref, o_ref):
    q = q_ref[...]
    for j in range(8):
        o_ref[:, 128 * j : 128 * (j + 1)] = jnp.dot(
            x_ref[:, 512 * j : 512 * (j + 1)],
            q,
            preferred_element_type=jnp.float32,
        )


def kernel(x):
    b, r, c = x.shape
    n = b * r
    flat = x.reshape(n, c)
    q = (jnp.arange(512)[:, None] == 4 * jnp.arange(128)[None, :]).astype(
        x.dtype
    )
    out = pl.pallas_call(
        _select_kernel,
        grid=(n // ROWS,),
        in_specs=[
            pl.BlockSpec((ROWS, c), lambda i: (i, 0)),
            pl.BlockSpec((512, 128), lambda i: (0, 0)),
        ],
        out_specs=pl.BlockSpec((ROWS, c // 4), lambda i: (i, 0)),
        out_shape=jax.ShapeDtypeStruct((n, c // 4), x.dtype),
    )(flat, q)
    return out.reshape(b, r, c // 4)
